# baseline (speedup 1.0000x reference)
import jax
import jax.numpy as jnp
import numpy as np
from jax.experimental import pallas as pl

_D = 128
_K_RATIO = 0.5


def _typed_linear(x, t, basis, coef):
    yb = jnp.einsum('mi,bio->mbo', x, basis)
    c = coef[t]
    return jnp.einsum('mb,mbo->mo', c, yb)


def kernel(h, edge_index, etype, basis, coef, Wq, bq, Wk, bk, Wv, bv):
    n = h.shape[0]
    scale = np.sqrt(_D).astype(np.float32)
    src = edge_index[0]
    dst = edge_index[1]
    self_emb = jnp.concatenate([h, h], axis=1)
    t_self = jnp.full((n,), 4, dtype=jnp.int32)
    self_y = _typed_linear(self_emb, t_self, basis, coef) @ Wv + bv
    n_q = h @ Wq + bq

    def direction(s, d):
        z2 = jnp.concatenate([h[s], h[d]], axis=1)
        ee = _typed_linear(z2, etype, basis, coef)
        n_k = ee @ Wk + bk
        n_v = ee @ Wv + bv
        score = jnp.sum(n_k * n_q[d], axis=-1, keepdims=True)
        score = jnp.exp(jnp.clip(score / scale, -10.0, 10.0))
        mask = (s != d).astype(score.dtype)[:, None]
        score = score * mask
        e_val = score * n_v
        y_sum = jnp.zeros((n, 1), dtype=h.dtype).at[d].add(e_val)
        z_sum = jnp.zeros((n, 1), dtype=h.dtype).at[d].add(score)
        return y_sum / (z_sum + 1e-6)

    in_y = direction(src, dst)
    out_y = direction(dst, src)
    y = in_y + out_y + self_y

    n_keep = max(int(_K_RATIO * n), 2)
    node_att = y[:, 0]
    _, topk_idx = jax.lax.top_k(node_att, n_keep)
    sorted_values = jnp.sort(topk_idx)
    sorted_indices = jnp.argsort(topk_idx)
    selected_y = y[sorted_indices]
    selected_h = h[sorted_indices]
    updated_h = selected_h * jax.nn.sigmoid(selected_y)
    return updated_h, sorted_values


# speedup vs baseline: 1.0127x; 1.0127x over previous
import jax
import jax.numpy as jnp
import numpy as np
from jax.experimental import pallas as pl
from jax.experimental.pallas import tpu as pltpu

_D = 128
_K_RATIO = 0.5


def _typed_linear(x, t, basis, coef):
    yb = jnp.einsum('mi,bio->mbo', x, basis)
    c = coef[t]
    return jnp.einsum('mb,mbo->mo', c, yb)


def _nq_body(h_ref, w_ref, b_ref, o_ref):
    o_ref[...] = jnp.dot(h_ref[...], w_ref[...],
                         preferred_element_type=jnp.float32) + b_ref[...]


def _nq_pallas(h, Wq, bq):
    n = h.shape[0]
    return pl.pallas_call(
        _nq_body,
        out_shape=jax.ShapeDtypeStruct((n, _D), jnp.float32),
    )(h, Wq, bq.reshape(1, _D))


def _gate_body(h_ref, y_ref, o_ref):
    o_ref[...] = h_ref[...] * jax.nn.sigmoid(y_ref[...])


def _gate_pallas(selected_h, selected_y):
    k = selected_h.shape[0]
    return pl.pallas_call(
        _gate_body,
        out_shape=jax.ShapeDtypeStruct((k, _D), jnp.float32),
    )(selected_h, selected_y)


def kernel(h, edge_index, etype, basis, coef, Wq, bq, Wk, bk, Wv, bv):
    n = h.shape[0]
    scale = np.sqrt(_D).astype(np.float32)
    src = edge_index[0]
    dst = edge_index[1]
    self_emb = jnp.concatenate([h, h], axis=1)
    t_self = jnp.full((n,), 4, dtype=jnp.int32)
    self_y = _typed_linear(self_emb, t_self, basis, coef) @ Wv + bv
    n_q = _nq_pallas(h, Wq, bq)

    def direction(s, d):
        z2 = jnp.concatenate([h[s], h[d]], axis=1)
        ee = _typed_linear(z2, etype, basis, coef)
        n_k = ee @ Wk + bk
        n_v = ee @ Wv + bv
        score = jnp.sum(n_k * n_q[d], axis=-1, keepdims=True)
        score = jnp.exp(jnp.clip(score / scale, -10.0, 10.0))
        mask = (s != d).astype(score.dtype)[:, None]
        score = score * mask
        e_val = score * n_v
        y_sum = jnp.zeros((n, 1), dtype=h.dtype).at[d].add(e_val)
        z_sum = jnp.zeros((n, 1), dtype=h.dtype).at[d].add(score)
        return y_sum / (z_sum + 1e-6)

    in_y = direction(src, dst)
    out_y = direction(dst, src)
    y = in_y + out_y + self_y

    n_keep = max(int(_K_RATIO * n), 2)
    node_att = y[:, 0]
    _, topk_idx = jax.lax.top_k(node_att, n_keep)
    sorted_values = jnp.sort(topk_idx)
    sorted_indices = jnp.argsort(topk_idx)
    selected_y = y[sorted_indices]
    selected_h = h[sorted_indices]
    updated_h = _gate_pallas(selected_h, selected_y)
    return updated_h, sorted_values


# speedup vs baseline: 1.1538x; 1.1394x over previous
import jax
import jax.numpy as jnp
import numpy as np
from jax.experimental import pallas as pl
from jax.experimental.pallas import tpu as pltpu

_D = 128
_K_RATIO = 0.5
_EB = 2560


def _nq_body(h_ref, w_ref, b_ref, o_ref):
    o_ref[...] = jnp.dot(h_ref[...], w_ref[...],
                         preferred_element_type=jnp.float32) + b_ref[...]


def _nq_pallas(h, Wq, bq):
    n = h.shape[0]
    return pl.pallas_call(
        _nq_body,
        out_shape=jax.ShapeDtypeStruct((n, _D), jnp.float32),
    )(h, Wq, bq.reshape(1, _D))


def _yb_self_body(h_ref, b2_ref, o_ref, z2_ref):
    h = h_ref[...]
    z2_ref[:, :_D] = h
    z2_ref[:, _D:] = h
    yb = jnp.dot(z2_ref[...], b2_ref[...], preferred_element_type=jnp.float32)
    o_ref[...] = yb.astype(jnp.bfloat16)


def _yb_self_pallas(h, B2):
    n = h.shape[0]
    return pl.pallas_call(
        _yb_self_body,
        grid=(5,),
        in_specs=[
            pl.BlockSpec((n // 5, _D), lambda i: (i, 0)),
            pl.BlockSpec((2 * _D, 2 * _D), lambda i: (0, 0)),
        ],
        out_specs=pl.BlockSpec((n // 5, 2 * _D), lambda i: (i, 0)),
        out_shape=jax.ShapeDtypeStruct((n, 2 * _D), jnp.bfloat16),
        scratch_shapes=[pltpu.VMEM((n // 5, 2 * _D), jnp.float32)],
    )(h, B2)


def _yb_edge_body(gs_ref, gd_ref, b2_ref, oin_ref, oout_ref, z2_ref):
    gs = gs_ref[...]
    gd = gd_ref[...]
    b2 = b2_ref[...]
    z2_ref[:, :_D] = gs
    z2_ref[:, _D:] = gd
    oin_ref[...] = jnp.dot(z2_ref[...], b2,
                           preferred_element_type=jnp.float32
                           ).astype(jnp.bfloat16)
    z2_ref[:, :_D] = gd
    z2_ref[:, _D:] = gs
    oout_ref[...] = jnp.dot(z2_ref[...], b2,
                            preferred_element_type=jnp.float32
                            ).astype(jnp.bfloat16)


def _yb_edge_pallas(Gs, Gd, B2):
    E = Gs.shape[0]
    blk = lambda i: (i, 0)
    return pl.pallas_call(
        _yb_edge_body,
        grid=(E // _EB,),
        in_specs=[
            pl.BlockSpec((_EB, _D), blk),
            pl.BlockSpec((_EB, _D), blk),
            pl.BlockSpec((2 * _D, 2 * _D), lambda i: (0, 0)),
        ],
        out_specs=[
            pl.BlockSpec((_EB, 2 * _D), blk),
            pl.BlockSpec((_EB, 2 * _D), blk),
        ],
        out_shape=[jax.ShapeDtypeStruct((E, 2 * _D), jnp.bfloat16)] * 2,
        scratch_shapes=[pltpu.VMEM((_EB, 2 * _D), jnp.float32)],
    )(Gs, Gd, B2)


def _gate_body(h_ref, y_ref, o_ref):
    o_ref[...] = h_ref[...] * jax.nn.sigmoid(y_ref[...])


def _gate_pallas(selected_h, selected_y):
    k = selected_h.shape[0]
    return pl.pallas_call(
        _gate_body,
        out_shape=jax.ShapeDtypeStruct((k, _D), jnp.float32),
    )(selected_h, selected_y)


def kernel(h, edge_index, etype, basis, coef, Wq, bq, Wk, bk, Wv, bv):
    n = h.shape[0]
    scale = np.sqrt(_D).astype(np.float32)
    src = edge_index[0]
    dst = edge_index[1]
    t_self = jnp.full((n,), 4, dtype=jnp.int32)
    B2 = jnp.transpose(basis, (1, 0, 2)).reshape(2 * _D, 2 * _D)

    yb_self = _yb_self_pallas(h, B2).reshape(n, 2, _D)
    ee_self = jnp.einsum('mb,mbo->mo', coef[t_self], yb_self)
    self_y = ee_self @ Wv + bv
    n_q = _nq_pallas(h, Wq, bq)

    Gs = h[src]
    Gd = h[dst]
    c_edge = coef[etype]
    yb_in, yb_out = _yb_edge_pallas(Gs, Gd, B2)

    def direction(s, d, yb):
        ee = jnp.einsum('mb,mbo->mo', c_edge, yb.reshape(-1, 2, _D))
        n_k = ee @ Wk + bk
        n_v = ee @ Wv + bv
        score = jnp.sum(n_k * n_q[d], axis=-1, keepdims=True)
        score = jnp.exp(jnp.clip(score / scale, -10.0, 10.0))
        mask = (s != d).astype(score.dtype)[:, None]
        score = score * mask
        e_val = score * n_v
        y_sum = jnp.zeros((n, 1), dtype=h.dtype).at[d].add(e_val)
        z_sum = jnp.zeros((n, 1), dtype=h.dtype).at[d].add(score)
        return y_sum / (z_sum + 1e-6)

    in_y = direction(src, dst, yb_in)
    out_y = direction(dst, src, yb_out)
    y = in_y + out_y + self_y

    n_keep = max(int(_K_RATIO * n), 2)
    node_att = y[:, 0]
    _, topk_idx = jax.lax.top_k(node_att, n_keep)
    sorted_values = jnp.sort(topk_idx)
    sorted_indices = jnp.argsort(topk_idx)
    selected_y = y[sorted_indices]
    selected_h = h[sorted_indices]
    updated_h = _gate_pallas(selected_h, selected_y)
    return updated_h, sorted_values
